# baseline (speedup 1.0000x reference)
import jax
import jax.numpy as jnp
from jax import lax
from jax.experimental import pallas as pl
from jax.experimental.pallas import tpu as pltpu
from jax.experimental.pallas import tpu_sc as plsc

_B = 16384
_F = 26
_V = 100000
_D = 32
_BC = 8192
_NB = _B // _BC
_T = _F * _NB
_GRP = _BC // (16 * 8)


def _body(x_hbm, tab_hbm, out_hbm, row_v, buf_v, rsem, isem, osem):
    d = lax.axis_index("s") * 2 + lax.axis_index("c")

    pltpu.async_copy(tab_hbm.at[0, d], row_v, rsem)
    pltpu.async_copy(x_hbm.at[0, pl.ds(0, _BC)], buf_v.at[0], isem)

    def step(t, carry):
        f = t // _NB
        c = lax.rem(t, _NB)

        pltpu.make_async_copy(
            x_hbm.at[0, pl.ds(0, _BC)], buf_v.at[c], isem
        ).wait()

        @pl.when(c == 0)
        def _():
            pltpu.make_async_copy(tab_hbm.at[0, d], row_v, rsem).wait()

        def gather8(i, carry2):
            base = i * 128
            for u in range(8):
                sl = pl.ds(base + u * 16, 16)
                iv = plsc.bitcast(buf_v[c, sl], jnp.int32)
                buf_v[c, sl] = plsc.load_gather(row_v, [iv])
            return carry2

        lax.fori_loop(0, _GRP, gather8, 0)

        @pl.when((c == _NB - 1) & (f + 1 < _F))
        def _():
            pltpu.async_copy(tab_hbm.at[f + 1, d], row_v, rsem)

        @pl.when(t >= 1)
        def _():
            pltpu.make_async_copy(
                x_hbm.at[0, pl.ds(0, _BC)], buf_v.at[1 - c], osem
            ).wait()

        @pl.when(t + 1 < _T)
        def _():
            t1 = t + 1
            f1 = t1 // _NB
            c1 = lax.rem(t1, _NB)
            pltpu.async_copy(
                x_hbm.at[f1, pl.ds(c1 * _BC, _BC)], buf_v.at[c1], isem
            )

        pltpu.async_copy(
            buf_v.at[c], out_hbm.at[f, d, pl.ds(c * _BC, _BC)], osem
        )
        return carry

    lax.fori_loop(0, _T, step, 0)

    pltpu.make_async_copy(
        x_hbm.at[0, pl.ds(0, _BC)], buf_v.at[1], osem
    ).wait()


@jax.jit
def kernel(x_cat, tables):
    x_t = jax.lax.bitcast_convert_type(x_cat.T, jnp.float32)
    tab_t = jnp.transpose(tables, (0, 2, 1))
    mesh = plsc.VectorSubcoreMesh(core_axis_name="c", subcore_axis_name="s")
    out = pl.kernel(
        _body,
        mesh=mesh,
        out_type=jax.ShapeDtypeStruct((_F, _D, _B), jnp.float32),
        scratch_types=[
            pltpu.VMEM((_V,), jnp.float32),
            pltpu.VMEM((_NB, _BC), jnp.float32),
            pltpu.SemaphoreType.DMA,
            pltpu.SemaphoreType.DMA,
            pltpu.SemaphoreType.DMA,
        ],
        compiler_params=pltpu.CompilerParams(
            use_tc_tiling_on_sc=True, needs_layout_passes=False
        ),
    )(x_t, tab_t)
    return jnp.transpose(out, (2, 0, 1))


# speedup vs baseline: 1.3908x; 1.3908x over previous
import jax
import jax.numpy as jnp
from jax import lax
from jax.experimental import pallas as pl
from jax.experimental.pallas import tpu as pltpu
from jax.experimental.pallas import tpu_sc as plsc

_B = 16384
_F = 26
_V = 100000
_D = 32
_GRP = _B // (16 * 8)


def _body(x_hbm, tab_hbm, out_hbm, row_v, buf_v, rsem, osem):
    d = lax.axis_index("s") * 2 + lax.axis_index("c")

    row_cp = pltpu.async_copy(tab_hbm.at[0, d], row_v, rsem)
    out_cp = None
    for f in range(_F):
        if out_cp is not None:
            out_cp.wait()
        pltpu.sync_copy(x_hbm.at[f], buf_v)
        row_cp.wait()

        def gather8(i, carry):
            base = i * 128
            for u in range(8):
                sl = pl.ds(base + u * 16, 16)
                iv = plsc.bitcast(buf_v[sl], jnp.int32)
                buf_v[sl] = plsc.load_gather(row_v, [iv])
            return carry

        lax.fori_loop(0, _GRP, gather8, 0)

        if f + 1 < _F:
            row_cp = pltpu.async_copy(tab_hbm.at[f + 1, d], row_v, rsem)
        out_cp = pltpu.async_copy(buf_v, out_hbm.at[f, d], osem)
    out_cp.wait()


@jax.jit
def kernel(x_cat, tables):
    x_t = jax.lax.bitcast_convert_type(x_cat.T, jnp.float32)
    tab_t = jnp.transpose(tables, (0, 2, 1))
    mesh = plsc.VectorSubcoreMesh(core_axis_name="c", subcore_axis_name="s")
    out = pl.kernel(
        _body,
        mesh=mesh,
        out_type=jax.ShapeDtypeStruct((_F, _D, _B), jnp.float32),
        scratch_types=[
            pltpu.VMEM((_V,), jnp.float32),
            pltpu.VMEM((_B,), jnp.float32),
            pltpu.SemaphoreType.DMA,
            pltpu.SemaphoreType.DMA,
        ],
        compiler_params=pltpu.CompilerParams(
            use_tc_tiling_on_sc=True, needs_layout_passes=False
        ),
    )(x_t, tab_t)
    return jnp.transpose(out, (2, 0, 1))


# speedup vs baseline: 1.8508x; 1.3308x over previous
import jax
import jax.numpy as jnp
from jax import lax
from jax.experimental import pallas as pl
from jax.experimental.pallas import tpu as pltpu
from jax.experimental.pallas import tpu_sc as plsc

_B = 16384
_F = 26
_V = 100000
_D = 32
_GRP = _B // (16 * 8)


def _body(x_hbm, tab_hbm, out_hbm, row_v, buf_v, rsem, osem):
    d = lax.axis_index("s") * 2 + lax.axis_index("c")

    row_cp = pltpu.async_copy(tab_hbm.at[0, d], row_v, rsem)
    out_cp = None
    for f in range(_F):
        if out_cp is not None:
            out_cp.wait()
        pltpu.sync_copy(x_hbm.at[f], buf_v)
        row_cp.wait()

        @plsc.parallel_loop(0, _B, step=16, unroll=8)
        def gather16(i):
            sl = pl.ds(i, 16)
            iv = plsc.bitcast(buf_v[sl], jnp.int32)
            buf_v[sl] = plsc.load_gather(row_v, [iv])

        if f + 1 < _F:
            row_cp = pltpu.async_copy(tab_hbm.at[f + 1, d], row_v, rsem)
        out_cp = pltpu.async_copy(buf_v, out_hbm.at[f, d], osem)
    out_cp.wait()


@jax.jit
def kernel(x_cat, tables):
    x_t = jax.lax.bitcast_convert_type(x_cat.T, jnp.float32)
    tab_t = jnp.transpose(tables, (0, 2, 1))
    mesh = plsc.VectorSubcoreMesh(core_axis_name="c", subcore_axis_name="s")
    out = pl.kernel(
        _body,
        mesh=mesh,
        out_type=jax.ShapeDtypeStruct((_F, _D, _B), jnp.float32),
        scratch_types=[
            pltpu.VMEM((_V,), jnp.float32),
            pltpu.VMEM((_B,), jnp.float32),
            pltpu.SemaphoreType.DMA,
            pltpu.SemaphoreType.DMA,
        ],
        compiler_params=pltpu.CompilerParams(
            use_tc_tiling_on_sc=True, needs_layout_passes=False
        ),
    )(x_t, tab_t)
    return jnp.transpose(out, (2, 0, 1))
